# Initial kernel scaffold; baseline (speedup 1.0000x reference)
import jax
import jax.numpy as jnp
from jax.experimental import pallas as pl


def kernel(w1_t, b1, w2_t, b2, x, neg_mask):
    raise NotImplementedError("write your pallas kernel here")


# speedup vs baseline: 7.0093x; 7.0093x over previous
import functools

import jax
import jax.numpy as jnp
from jax.experimental import pallas as pl
from jax.experimental.pallas import tpu as pltpu

C = 128
N = 8192
T_TILE = 256
S_TILE = 128
NEG_INF = float("-inf")


def _linear_kernel(w_ref, x_ref, b_ref, o_ref):
    h = jnp.dot(w_ref[...], x_ref[...], preferred_element_type=jnp.float32)
    o_ref[...] = (h + b_ref[...]).astype(jnp.bfloat16)


def _linear_t(w, x_t, b_col):
    tile = min(1024, N)
    return pl.pallas_call(
        _linear_kernel,
        out_shape=jax.ShapeDtypeStruct((C, N), jnp.bfloat16),
        grid=(N // tile,),
        in_specs=[
            pl.BlockSpec((C, C), lambda i: (0, 0)),
            pl.BlockSpec((C, tile), lambda i: (0, i)),
            pl.BlockSpec((C, 1), lambda i: (0, 0)),
        ],
        out_specs=pl.BlockSpec((C, tile), lambda i: (0, i)),
        compiler_params=pltpu.CompilerParams(
            dimension_semantics=("parallel",)),
    )(w, x_t, b_col)


def _agg_body(acc, mask_blk, h_blk):
    for s in range(S_TILE):
        hcol = jax.lax.slice(h_blk, (0, s), (C, s + 1))
        hb = jax.lax.broadcast_in_dim(hcol, (C, T_TILE), (0, 1))
        acc = jnp.maximum(acc, hb + mask_blk[s:s + 1, :])
    return acc


def _agg_lin_kernel(mask_ref, h_ref, w_ref, b_ref, o_ref, acc_ref):
    sc = pl.program_id(1)

    @pl.when(sc == 0)
    def _init():
        acc_ref[...] = jnp.full_like(acc_ref, NEG_INF)

    acc_ref[...] = _agg_body(acc_ref[...], mask_ref[...], h_ref[...])

    @pl.when(sc == pl.num_programs(1) - 1)
    def _finalize():
        a = acc_ref[...]
        a = jnp.where(a > NEG_INF, a, jnp.bfloat16(0.0))
        a = jnp.maximum(a, jnp.bfloat16(0.0))
        h2 = jnp.dot(w_ref[...], a, preferred_element_type=jnp.float32)
        o_ref[...] = (h2 + b_ref[...]).astype(jnp.bfloat16)


def _agg_out_kernel(mask_ref, h_ref, o_ref, acc_ref):
    sc = pl.program_id(1)

    @pl.when(sc == 0)
    def _init():
        acc_ref[...] = jnp.full_like(acc_ref, NEG_INF)

    acc_ref[...] = _agg_body(acc_ref[...], mask_ref[...], h_ref[...])

    @pl.when(sc == pl.num_programs(1) - 1)
    def _finalize():
        a = acc_ref[...]
        o_ref[...] = jnp.where(a > NEG_INF, a, jnp.bfloat16(0.0)
                               ).astype(jnp.float32)


def _agg_grid_specs():
    return dict(
        grid=(N // T_TILE, N // S_TILE),
        scratch_shapes=[pltpu.VMEM((C, T_TILE), jnp.bfloat16)],
        compiler_params=pltpu.CompilerParams(
            dimension_semantics=("parallel", "arbitrary")),
    )


def _agg_linear(mask_t, h_t, w, b_col):
    return pl.pallas_call(
        _agg_lin_kernel,
        out_shape=jax.ShapeDtypeStruct((C, N), jnp.bfloat16),
        in_specs=[
            pl.BlockSpec((S_TILE, T_TILE), lambda tb, sc: (sc, tb)),
            pl.BlockSpec((C, S_TILE), lambda tb, sc: (0, sc)),
            pl.BlockSpec((C, C), lambda tb, sc: (0, 0)),
            pl.BlockSpec((C, 1), lambda tb, sc: (0, 0)),
        ],
        out_specs=pl.BlockSpec((C, T_TILE), lambda tb, sc: (0, tb)),
        **_agg_grid_specs(),
    )(mask_t, h_t, w, b_col)


def _agg_final(mask_t, h_t):
    return pl.pallas_call(
        _agg_out_kernel,
        out_shape=jax.ShapeDtypeStruct((C, N), jnp.float32),
        in_specs=[
            pl.BlockSpec((S_TILE, T_TILE), lambda tb, sc: (sc, tb)),
            pl.BlockSpec((C, S_TILE), lambda tb, sc: (0, sc)),
        ],
        out_specs=pl.BlockSpec((C, T_TILE), lambda tb, sc: (0, tb)),
        **_agg_grid_specs(),
    )(mask_t, h_t)


def kernel(w1_t, b1, w2_t, b2, x, neg_mask):
    mask_t = neg_mask.T
    x_t = x.T.astype(jnp.bfloat16)
    w1 = w1_t.T
    w2 = w2_t.T
    b1_col = b1.T
    b2_col = b2.T

    h1_t = _linear_t(w1, x_t, b1_col)
    h2_t = _agg_linear(mask_t, h1_t, w2, b2_col)
    a2_t = _agg_final(mask_t, h2_t)
    return a2_t.T


# speedup vs baseline: 13.7934x; 1.9679x over previous
import jax
import jax.numpy as jnp
from jax.experimental import pallas as pl
from jax.experimental.pallas import tpu as pltpu

C = 128
N = 8192
T_TILE = 512
S_CHUNK = 256
TB_PER_CORE = 8
NEG_INF = float("-inf")


def _linear_kernel(w_ref, x_ref, b_ref, o_ref):
    h = jnp.dot(w_ref[...], x_ref[...], preferred_element_type=jnp.float32)
    o_ref[...] = (h + b_ref[...]).astype(jnp.bfloat16)


def _linear_t(w, x_t, b_col):
    tile = min(1024, N)
    return pl.pallas_call(
        _linear_kernel,
        out_shape=jax.ShapeDtypeStruct((C, N), jnp.bfloat16),
        grid=(N // tile,),
        in_specs=[
            pl.BlockSpec((C, C), lambda i: (0, 0)),
            pl.BlockSpec((C, tile), lambda i: (0, i)),
            pl.BlockSpec((C, 1), lambda i: (0, 0)),
        ],
        out_specs=pl.BlockSpec((C, tile), lambda i: (0, i)),
        compiler_params=pltpu.CompilerParams(
            dimension_semantics=("parallel",)),
    )(w, x_t, b_col)


def _build_bcast(h_ref, hbc_ref):
    h_blk = h_ref[...]
    for s in range(S_CHUNK):
        col = jax.lax.slice(h_blk, (0, s), (C, s + 1))
        hbc_ref[s] = jax.lax.broadcast_in_dim(col, (C, T_TILE), (0, 1))


def _accumulate(acc, mask_blk, hbc_ref):
    for s in range(S_CHUNK):
        acc = jnp.maximum(acc, hbc_ref[s] + mask_blk[s:s + 1, :])
    return acc


def _agg_lin_kernel(mask_ref, h_ref, w_ref, b_ref, o_ref, hbc_ref, acc_ref):
    sc = pl.program_id(1)
    tbi = pl.program_id(2)

    @pl.when(tbi == 0)
    def _build():
        _build_bcast(h_ref, hbc_ref)

    @pl.when(sc == 0)
    def _init():
        acc_ref[tbi] = jnp.full((C, T_TILE), NEG_INF, jnp.bfloat16)

    acc_ref[tbi] = _accumulate(acc_ref[tbi], mask_ref[...], hbc_ref)

    @pl.when(sc == pl.num_programs(1) - 1)
    def _finalize():
        a = acc_ref[tbi]
        a = jnp.where(a > NEG_INF, a, jnp.bfloat16(0.0))
        a = jnp.maximum(a, jnp.bfloat16(0.0))
        h2 = jnp.dot(w_ref[...], a, preferred_element_type=jnp.float32)
        o_ref[...] = (h2 + b_ref[...]).astype(jnp.bfloat16)


def _agg_out_kernel(mask_ref, h_ref, o_ref, hbc_ref, acc_ref):
    sc = pl.program_id(1)
    tbi = pl.program_id(2)

    @pl.when(tbi == 0)
    def _build():
        _build_bcast(h_ref, hbc_ref)

    @pl.when(sc == 0)
    def _init():
        acc_ref[tbi] = jnp.full((C, T_TILE), NEG_INF, jnp.bfloat16)

    acc_ref[tbi] = _accumulate(acc_ref[tbi], mask_ref[...], hbc_ref)

    @pl.when(sc == pl.num_programs(1) - 1)
    def _finalize():
        a = acc_ref[tbi]
        o_ref[...] = jnp.where(a > NEG_INF, a, jnp.bfloat16(0.0)
                               ).astype(jnp.float32)


def _agg_grid_specs():
    return dict(
        grid=(2, N // S_CHUNK, TB_PER_CORE),
        scratch_shapes=[
            pltpu.VMEM((S_CHUNK, C, T_TILE), jnp.bfloat16),
            pltpu.VMEM((TB_PER_CORE, C, T_TILE), jnp.bfloat16),
        ],
        compiler_params=pltpu.CompilerParams(
            dimension_semantics=("parallel", "arbitrary", "arbitrary")),
    )


def _mask_spec():
    return pl.BlockSpec(
        (S_CHUNK, T_TILE),
        lambda tbo, sc, tbi: (sc, tbo * TB_PER_CORE + tbi))


def _out_spec():
    last = N // S_CHUNK - 1
    return pl.BlockSpec(
        (C, T_TILE),
        lambda tbo, sc, tbi: (
            0,
            jnp.where(sc == last, tbo * TB_PER_CORE + tbi,
                      tbo * TB_PER_CORE)))


def _agg_linear(mask_t, h_t, w, b_col):
    return pl.pallas_call(
        _agg_lin_kernel,
        out_shape=jax.ShapeDtypeStruct((C, N), jnp.bfloat16),
        in_specs=[
            _mask_spec(),
            pl.BlockSpec((C, S_CHUNK), lambda tbo, sc, tbi: (0, sc)),
            pl.BlockSpec((C, C), lambda tbo, sc, tbi: (0, 0)),
            pl.BlockSpec((C, 1), lambda tbo, sc, tbi: (0, 0)),
        ],
        out_specs=_out_spec(),
        **_agg_grid_specs(),
    )(mask_t, h_t, w, b_col)


def _agg_final(mask_t, h_t):
    return pl.pallas_call(
        _agg_out_kernel,
        out_shape=jax.ShapeDtypeStruct((C, N), jnp.float32),
        in_specs=[
            _mask_spec(),
            pl.BlockSpec((C, S_CHUNK), lambda tbo, sc, tbi: (0, sc)),
        ],
        out_specs=_out_spec(),
        **_agg_grid_specs(),
    )(mask_t, h_t)


def kernel(w1_t, b1, w2_t, b2, x, neg_mask):
    mask_t = neg_mask.T
    x_t = x.T.astype(jnp.bfloat16)
    w1 = w1_t.T
    w2 = w2_t.T
    b1_col = b1.T
    b2_col = b2.T

    h1_t = _linear_t(w1, x_t, b1_col)
    h2_t = _agg_linear(mask_t, h1_t, w2, b2_col)
    a2_t = _agg_final(mask_t, h2_t)
    return a2_t.T
